# baseline (speedup 1.0000x reference)
import functools

import jax
import jax.numpy as jnp
from jax import lax
from jax.experimental import pallas as pl
from jax.experimental.pallas import tpu as pltpu
from jax.experimental.pallas import tpu_sc as plsc

EMBED = 64
IDXW = 128
GATHERS_PER_CHUNK = 4
CHUNK = GATHERS_PER_CHUNK * IDXW


def _emb_body(idx_hbm, table_hbm, out_hbm, idx_v, rows_v, gsem, n_chunks):
    nc = 2
    wid = lax.axis_index("s") * nc + lax.axis_index("c")
    base_w = wid * (n_chunks * CHUNK)

    def chunk_body(i, carry):
        base = pl.multiple_of(base_w + i * CHUNK, CHUNK)
        pltpu.sync_copy(idx_hbm.at[pl.ds(base, CHUNK)], idx_v)
        copies = []
        for j in range(GATHERS_PER_CHUNK):
            copies.append(
                pltpu.async_copy(
                    table_hbm.at[idx_v.at[pl.ds(j * IDXW, IDXW)]],
                    rows_v.at[pl.ds(j * IDXW, IDXW)],
                    gsem,
                )
            )
        for c in copies:
            c.wait()
        pltpu.sync_copy(rows_v, out_hbm.at[pl.ds(base, CHUNK)])
        return carry

    lax.fori_loop(0, n_chunks, chunk_body, 0)


def kernel(tokens_ids, table):
    batch, seq = tokens_ids.shape
    vocab, embed = table.shape
    n = batch * seq
    nw = 32
    assert embed == EMBED and n % (nw * CHUNK) == 0
    n_chunks = n // (nw * CHUNK)

    flat_ids = tokens_ids.reshape(n)

    grid_kernel = pl.kernel(
        functools.partial(_emb_body, n_chunks=n_chunks),
        out_type=jax.ShapeDtypeStruct((n, embed), jnp.float32),
        mesh=plsc.VectorSubcoreMesh(core_axis_name="c", subcore_axis_name="s"),
        scratch_types=[
            pltpu.VMEM((CHUNK,), jnp.int32),
            pltpu.VMEM((CHUNK, EMBED), jnp.float32),
            pltpu.SemaphoreType.DMA,
        ],
        compiler_params=pltpu.CompilerParams(use_tc_tiling_on_sc=False),
    )
    out = grid_kernel(flat_ids, table)
    return out.reshape(batch, seq, embed)


# speedup vs baseline: 1.0433x; 1.0433x over previous
import functools

import jax
import jax.numpy as jnp
from jax import lax
from jax.experimental import pallas as pl
from jax.experimental.pallas import tpu as pltpu
from jax.experimental.pallas import tpu_sc as plsc

EMBED = 64
IDXW = 128
GATHERS_PER_CHUNK = 5
CHUNK = GATHERS_PER_CHUNK * IDXW
NSLOT = 2


def _emb_body(idx_hbm, table_hbm, out_hbm, idx_v, rows_v,
              isem0, isem1, wsem0, wsem1, gsem, n_chunks):
    isems = (isem0, isem1)
    wsems = (wsem0, wsem1)
    nc = 2
    wid = lax.axis_index("s") * nc + lax.axis_index("c")
    base_w = wid * (n_chunks * CHUNK)
    n_pairs = n_chunks // NSLOT

    def cbase(i):
        return pl.multiple_of(base_w + i * CHUNK, 8)

    def idx_start(i, b):
        pltpu.async_copy(idx_hbm.at[pl.ds(cbase(i), CHUNK)], idx_v.at[b],
                         isems[b])

    def idx_wait(b):
        pltpu.make_async_copy(idx_hbm.at[pl.ds(0, CHUNK)], idx_v.at[b],
                              isems[b]).wait()

    def gathers(b):
        cs = []
        for j in range(GATHERS_PER_CHUNK):
            cs.append(pltpu.async_copy(
                table_hbm.at[idx_v.at[b, pl.ds(j * IDXW, IDXW)]],
                rows_v.at[b, pl.ds(j * IDXW, IDXW)],
                gsem))
        for c in cs:
            c.wait()

    def wb_start(i, b):
        pltpu.async_copy(rows_v.at[b], out_hbm.at[pl.ds(cbase(i), CHUNK)],
                         wsems[b])

    def wb_wait(b):
        pltpu.make_async_copy(rows_v.at[b], out_hbm.at[pl.ds(0, CHUNK)],
                              wsems[b]).wait()

    idx_start(0, 0)
    idx_start(1, 1)

    for b in range(NSLOT):
        idx_wait(b)
        gathers(b)
        idx_start(b + NSLOT, b)
        wb_start(b, b)

    def pair_body(p, carry):
        for b in range(NSLOT):
            i = NSLOT * p + b
            wb_wait(b)
            idx_wait(b)
            gathers(b)
            idx_start(i + NSLOT, b)
            wb_start(i, b)
        return carry

    lax.fori_loop(1, n_pairs - 1, pair_body, 0)

    for b in range(NSLOT):
        i = NSLOT * (n_pairs - 1) + b
        wb_wait(b)
        idx_wait(b)
        gathers(b)
        wb_start(i, b)
    for b in range(NSLOT):
        wb_wait(b)


def kernel(tokens_ids, table):
    batch, seq = tokens_ids.shape
    vocab, embed = table.shape
    n = batch * seq
    nw = 32
    assert embed == EMBED and n % (nw * CHUNK) == 0
    n_chunks = n // (nw * CHUNK)
    assert n_chunks % NSLOT == 0 and n_chunks >= 2 * NSLOT

    flat_ids = tokens_ids.reshape(n)

    grid_kernel = pl.kernel(
        functools.partial(_emb_body, n_chunks=n_chunks),
        out_type=jax.ShapeDtypeStruct((n, embed), jnp.float32),
        mesh=plsc.VectorSubcoreMesh(core_axis_name="c", subcore_axis_name="s"),
        scratch_types=[
            pltpu.VMEM((NSLOT, CHUNK), jnp.int32),
            pltpu.VMEM((NSLOT, CHUNK, EMBED), jnp.float32),
            pltpu.SemaphoreType.DMA,
            pltpu.SemaphoreType.DMA,
            pltpu.SemaphoreType.DMA,
            pltpu.SemaphoreType.DMA,
            pltpu.SemaphoreType.DMA,
        ],
        compiler_params=pltpu.CompilerParams(use_tc_tiling_on_sc=False),
    )
    out = grid_kernel(flat_ids, table)
    return out.reshape(batch, seq, embed)
